# Initial kernel scaffold; baseline (speedup 1.0000x reference)
import jax
import jax.numpy as jnp
from jax.experimental import pallas as pl


def kernel(V, E, edges, fe_W1, fe_b1, fe_W2, fe_b2, fn_W1, fn_b1, fn_W2, fn_b2):
    raise NotImplementedError("write your pallas kernel here")


# speedup vs baseline: 7.2746x; 7.2746x over previous
import functools

import jax
import jax.numpy as jnp
from jax import lax
from jax.experimental import pallas as pl
from jax.experimental.pallas import tpu as pltpu
from jax.experimental.pallas import tpu_sc as plsc

B, N, M = 4, 10000, 160000
D = 128
NW = 32
EPW = (B * M) // NW
CE = 80
NCH = EPW // CE
CR = 160
RPS = (2 * M) // 16
NSC = RPS // CR
NP = 20480
PSUB = NP // 16


def _pre_body(v_ref, w_ref, o_ref):
    o_ref[0, 0] = jnp.dot(v_ref[0], w_ref[0],
                          preferred_element_type=jnp.float32)


def _edge_body(g_ref, e_ref, w1_ref, b1_ref, w2_ref, b2_ref, o_ref):
    x = g_ref[...] + jnp.dot(e_ref[...], w1_ref[...],
                             preferred_element_type=jnp.float32) + b1_ref[...]
    h = x * jax.nn.sigmoid(x)
    o_ref[...] = jnp.dot(h, w2_ref[...],
                         preferred_element_type=jnp.float32) + b2_ref[...]


def _node_body(v_ref, s0_ref, s1_ref, wv_ref, wm0_ref,
               wm1_ref, b1_ref, w2_ref, b2_ref, o_ref):
    s0 = s0_ref[0, 0]
    s1 = s1_ref[0, 0]
    m0 = s0[:, :64] / jnp.maximum(s0[:, 64:65], 1.0)
    m1 = s1[:, 64:] / jnp.maximum(s1[:, 0:1], 1.0)
    x = (jnp.dot(v_ref[0], wv_ref[...], preferred_element_type=jnp.float32)
         + jnp.dot(m0, wm0_ref[...], preferred_element_type=jnp.float32)
         + jnp.dot(m1, wm1_ref[...], preferred_element_type=jnp.float32)
         + b1_ref[...])
    h = x * jax.nn.sigmoid(x)
    o_ref[0] = jnp.dot(h, w2_ref[...],
                       preferred_element_type=jnp.float32) + b2_ref[...]


def _gather_body(table, edges, out, ebuf, idx2, g_a, g_b, obuf, sem):
    c = lax.axis_index("c")
    s = lax.axis_index("s")
    wid = c * 16 + s
    e_base = wid * EPW
    b = e_base // M
    off = (lax.iota(jnp.int32, 16) & 1) * N + b * (2 * N)

    def chunk(i, carry):
        be = e_base + i * CE
        pltpu.sync_copy(edges.at[pl.ds(2 * be, 2 * CE)], ebuf)
        for t in range(2 * CE // 16):
            v = ebuf[pl.ds(t * 16, 16)] + off
            idx2[(t * 16) // CE, pl.ds((t * 16) % CE, 16)] = v
        d1 = pltpu.async_copy(table.at[idx2.at[0]], g_a, sem)
        d2 = pltpu.async_copy(table.at[idx2.at[1]], g_b, sem)
        d1.wait()
        d2.wait()

        def pair_add(j, carry2):
            for k in range(8):
                sl = pl.ds(k * 16, 16)
                obuf[j, sl] = g_a[2 * j, sl] + g_a[2 * j + 1, sl]
                obuf[CE // 2 + j, sl] = g_b[2 * j, sl] + g_b[2 * j + 1, sl]
            return carry2

        lax.fori_loop(0, CE // 2, pair_add, 0)
        pltpu.sync_copy(obuf, out.at[pl.ds(be, CE)])
        return carry

    lax.fori_loop(0, NCH, chunk, 0)


BCH = 64
NP1 = 10240
PS1 = NP1 // 16
CH = 80
NCH2 = (M // 16) // CH


def _scatter_body(ee, edges_t, z64, out, acc, vbuf, idxb, zb, ob_buf):
    c = lax.axis_index("c")
    s = lax.axis_index("s")
    coff = (1 - c) * 64
    onehot = jnp.where(lax.iota(jnp.int32, 16) == 0,
                       jnp.float32(1.0), jnp.float32(0.0))
    pltpu.sync_copy(z64, zb)
    for b in range(B):

        def zero_k(k, carry):
            pltpu.sync_copy(zb, acc.at[pl.ds(s * PS1 + k * BCH, BCH)])
            return carry

        lax.fori_loop(0, PS1 // BCH, zero_k, 0)
        plsc.subcore_barrier()
        eb0 = b * M + s * (M // 16)

        def chunk(i, carry):
            eb = eb0 + i * CH
            pltpu.sync_copy(ee.at[pl.ds(eb, CH)], vbuf)
            pltpu.sync_copy(edges_t.at[pl.ds((2 * b + c) * M + s * (M // 16)
                                             + i * CH, CH)], idxb)

            def mark(j, carry2):
                for u in range(8):
                    vbuf[j * 8 + u, pl.ds(coff, 16)] = onehot
                return carry2

            lax.fori_loop(0, CH // 8, mark, 0)
            pltpu.sync_copy(vbuf, acc.at[idxb], add=True)
            return carry

        lax.fori_loop(0, NCH2, chunk, 0)
        plsc.subcore_barrier()
        base_o = (c * B + b) * NP1 + s * PS1

        def out_k(k, carry):
            pltpu.sync_copy(acc.at[pl.ds(s * PS1 + k * BCH, BCH)], ob_buf)
            pltpu.sync_copy(ob_buf, out.at[pl.ds(base_o + k * BCH, BCH)])
            return carry

        lax.fori_loop(0, PS1 // BCH, out_k, 0)
        plsc.subcore_barrier()


def _sc_mesh():
    return plsc.VectorSubcoreMesh(core_axis_name="c", subcore_axis_name="s")


def _gather_call(table, edges_flat):
    return pl.kernel(
        _gather_body,
        out_type=jax.ShapeDtypeStruct((B * M, D), jnp.float32),
        mesh=_sc_mesh(),
        scratch_types=[
            pltpu.VMEM((2 * CE,), jnp.int32),
            pltpu.VMEM((2, CE), jnp.int32),
            pltpu.VMEM((CE, D), jnp.float32),
            pltpu.VMEM((CE, D), jnp.float32),
            pltpu.VMEM((CE, D), jnp.float32),
            pltpu.SemaphoreType.DMA,
        ],
    )(table, edges_flat)


def _scatter_call(ee, edges_flat, z64):
    return pl.kernel(
        _scatter_body,
        out_type=jax.ShapeDtypeStruct((2 * B * NP1, D), jnp.float32),
        mesh=_sc_mesh(),
        scratch_types=[
            pltpu.VMEM_SHARED((NP1, D), jnp.float32),
            pltpu.VMEM((CH, D), jnp.float32),
            pltpu.VMEM((CH,), jnp.int32),
            pltpu.VMEM((BCH, D), jnp.float32),
            pltpu.VMEM((BCH, D), jnp.float32),
        ],
    )(ee, edges_flat, z64)


def kernel(V, E, edges, fe_W1, fe_b1, fe_W2, fe_b2,
           fn_W1, fn_b1, fn_W2, fn_b2):
    f32 = jnp.float32
    w_sr = fe_W1[:256].reshape(2, D, D)
    w1e = fe_W1[256:]
    b1e = fe_b1.reshape(1, D)
    b2e = fe_b2.reshape(1, D)
    wv, wm0, wm1 = fn_W1[:128], fn_W1[128:192], fn_W1[192:256]
    b1n = fn_b1.reshape(1, D)
    b2n = fn_b2.reshape(1, D)

    tn = 2000
    table = pl.pallas_call(
        _pre_body,
        grid=(B, 2, N // tn),
        in_specs=[
            pl.BlockSpec((1, tn, D), lambda b, p, i: (b, i, 0)),
            pl.BlockSpec((1, D, D), lambda b, p, i: (p, 0, 0)),
        ],
        out_specs=pl.BlockSpec((1, 1, tn, D), lambda b, p, i: (b, p, i, 0)),
        out_shape=jax.ShapeDtypeStruct((B, 2, N, D), f32),
    )(V, w_sr).reshape(B * 2 * N, D)

    edges_flat = edges.reshape(B * 2 * M)

    G = _gather_call(table, edges_flat)

    tm = 1024
    EE = pl.pallas_call(
        _edge_body,
        grid=(B * M // tm,),
        in_specs=[
            pl.BlockSpec((tm, D), lambda i: (i, 0)),
            pl.BlockSpec((tm, D), lambda i: (i, 0)),
            pl.BlockSpec((D, D), lambda i: (0, 0)),
            pl.BlockSpec((1, D), lambda i: (0, 0)),
            pl.BlockSpec((D, D), lambda i: (0, 0)),
            pl.BlockSpec((1, D), lambda i: (0, 0)),
        ],
        out_specs=pl.BlockSpec((tm, D), lambda i: (i, 0)),
        out_shape=jax.ShapeDtypeStruct((B * M, D), f32),
    )(G, E.reshape(B * M, D), w1e, b1e, fe_W2, b2e)

    z64 = jnp.zeros((BCH, D), f32)
    edges_t = edges.transpose(0, 2, 1).reshape(B * 2 * M)
    accs = _scatter_call(EE, edges_t, z64).reshape(2, B, NP1, D)

    node_emb = _node_call(V, accs, wv, wm0, wm1, b1n, fn_W2, b2n)

    return (node_emb, EE.reshape(B, M, D))


def _node_call(V, accs, wv, wm0, wm1, b1n, w2, b2n):
    tno = 1000
    return pl.pallas_call(
        _node_body,
        grid=(B, N // tno),
        in_specs=[
            pl.BlockSpec((1, tno, D), lambda b, i: (b, i, 0)),
            pl.BlockSpec((1, 1, tno, D), lambda b, i: (0, b, i, 0)),
            pl.BlockSpec((1, 1, tno, D), lambda b, i: (1, b, i, 0)),
            pl.BlockSpec((D, D), lambda b, i: (0, 0)),
            pl.BlockSpec((64, D), lambda b, i: (0, 0)),
            pl.BlockSpec((64, D), lambda b, i: (0, 0)),
            pl.BlockSpec((1, D), lambda b, i: (0, 0)),
            pl.BlockSpec((D, D), lambda b, i: (0, 0)),
            pl.BlockSpec((1, D), lambda b, i: (0, 0)),
        ],
        out_specs=pl.BlockSpec((1, tno, D), lambda b, i: (b, i, 0)),
        out_shape=jax.ShapeDtypeStruct((B, N, D), jnp.float32),
    )(V, accs, accs, wv, wm0, wm1, b1n, w2, b2n)


# speedup vs baseline: 8.1637x; 1.1222x over previous
import functools

import jax
import jax.numpy as jnp
from jax import lax
from jax.experimental import pallas as pl
from jax.experimental.pallas import tpu as pltpu
from jax.experimental.pallas import tpu_sc as plsc

B, N, M = 4, 10000, 160000
D = 128
NW = 32
EPW = (B * M) // NW
CE = 80
NCH = EPW // CE
CR = 160
RPS = (2 * M) // 16
NSC = RPS // CR
NP = 20480
PSUB = NP // 16


def _pre_body(v_ref, w_ref, o_ref):
    o_ref[0, 0] = jnp.dot(v_ref[0], w_ref[0],
                          preferred_element_type=jnp.float32)


def _edge_body(g_ref, e_ref, w1_ref, b1_ref, w2_ref, b2_ref, o_ref):
    x = g_ref[...] + jnp.dot(e_ref[...], w1_ref[...],
                             preferred_element_type=jnp.float32) + b1_ref[...]
    h = x * jax.nn.sigmoid(x)
    o_ref[...] = jnp.dot(h, w2_ref[...],
                         preferred_element_type=jnp.float32) + b2_ref[...]


def _node_body(v_ref, s0_ref, s1_ref, wv_ref, wm0_ref,
               wm1_ref, b1_ref, w2_ref, b2_ref, o_ref):
    s0 = s0_ref[0, 0]
    s1 = s1_ref[0, 0]
    m0 = s0[:, :64] / jnp.maximum(s0[:, 64:65], 1.0)
    m1 = s1[:, 64:] / jnp.maximum(s1[:, 0:1], 1.0)
    x = (jnp.dot(v_ref[0], wv_ref[...], preferred_element_type=jnp.float32)
         + jnp.dot(m0, wm0_ref[...], preferred_element_type=jnp.float32)
         + jnp.dot(m1, wm1_ref[...], preferred_element_type=jnp.float32)
         + b1_ref[...])
    h = x * jax.nn.sigmoid(x)
    o_ref[0] = jnp.dot(h, w2_ref[...],
                       preferred_element_type=jnp.float32) + b2_ref[...]


def _gather_body(table, edges, out, ebuf0, ebuf1, idx0, idx1,
                 ga0, gb0, ga1, gb1, obuf, sem0, sem1):
    c = lax.axis_index("c")
    s = lax.axis_index("s")
    wid = c * 16 + s
    e_base = wid * EPW
    b = e_base // M
    off = (lax.iota(jnp.int32, 16) & 1) * N + b * (2 * N)

    def stage(i, ebuf, idx2, g_a, g_b, sem):
        be = e_base + i * CE
        pltpu.sync_copy(edges.at[pl.ds(2 * be, 2 * CE)], ebuf)
        for t in range(2 * CE // 16):
            v = ebuf[pl.ds(t * 16, 16)] + off
            idx2[(t * 16) // CE, pl.ds((t * 16) % CE, 16)] = v
        pltpu.async_copy(table.at[idx2.at[0]], g_a, sem)
        pltpu.async_copy(table.at[idx2.at[1]], g_b, sem)

    def drain(i, g_a, g_b, sem):
        be = e_base + i * CE
        pltpu.make_async_copy(table.at[idx0.at[0]], g_a, sem).wait()
        pltpu.make_async_copy(table.at[idx0.at[0]], g_b, sem).wait()

        def pair_add(j, carry2):
            for k in range(8):
                sl = pl.ds(k * 16, 16)
                obuf[j, sl] = g_a[2 * j, sl] + g_a[2 * j + 1, sl]
                obuf[CE // 2 + j, sl] = g_b[2 * j, sl] + g_b[2 * j + 1, sl]
            return carry2

        lax.fori_loop(0, CE // 2, pair_add, 0, unroll=4)
        pltpu.sync_copy(obuf, out.at[pl.ds(be, CE)])

    stage(0, ebuf0, idx0, ga0, gb0, sem0)

    def body(k, carry):
        i0 = 2 * k
        stage(i0 + 1, ebuf1, idx1, ga1, gb1, sem1)
        drain(i0, ga0, gb0, sem0)

        @pl.when(k + 1 < NCH // 2)
        def _():
            stage(i0 + 2, ebuf0, idx0, ga0, gb0, sem0)

        drain(i0 + 1, ga1, gb1, sem1)
        return carry

    lax.fori_loop(0, NCH // 2, body, 0)


BCH = 64
NP1 = 10240
PS1 = NP1 // 16
CH = 80
NCH2 = (M // 16) // CH


def _scatter_body(ee, edges_t, z64, out, acc, vbuf, idxb, zb, ob_buf):
    c = lax.axis_index("c")
    s = lax.axis_index("s")
    coff = (1 - c) * 64
    onehot = jnp.where(lax.iota(jnp.int32, 16) == 0,
                       jnp.float32(1.0), jnp.float32(0.0))
    pltpu.sync_copy(z64, zb)
    for b in range(B):

        def zero_k(k, carry):
            pltpu.sync_copy(zb, acc.at[pl.ds(s * PS1 + k * BCH, BCH)])
            return carry

        lax.fori_loop(0, PS1 // BCH, zero_k, 0)
        plsc.subcore_barrier()
        eb0 = b * M + s * (M // 16)

        def chunk(i, carry):
            eb = eb0 + i * CH
            pltpu.sync_copy(ee.at[pl.ds(eb, CH)], vbuf)
            pltpu.sync_copy(edges_t.at[pl.ds((2 * b + c) * M + s * (M // 16)
                                             + i * CH, CH)], idxb)

            def mark(j, carry2):
                for u in range(8):
                    vbuf[j * 8 + u, pl.ds(coff, 16)] = onehot
                return carry2

            lax.fori_loop(0, CH // 8, mark, 0)
            pltpu.sync_copy(vbuf, acc.at[idxb], add=True)
            return carry

        lax.fori_loop(0, NCH2, chunk, 0)
        plsc.subcore_barrier()
        base_o = (c * B + b) * NP1 + s * PS1

        def out_k(k, carry):
            pltpu.sync_copy(acc.at[pl.ds(s * PS1 + k * BCH, BCH)], ob_buf)
            pltpu.sync_copy(ob_buf, out.at[pl.ds(base_o + k * BCH, BCH)])
            return carry

        lax.fori_loop(0, PS1 // BCH, out_k, 0)
        plsc.subcore_barrier()


def _sc_mesh():
    return plsc.VectorSubcoreMesh(core_axis_name="c", subcore_axis_name="s")


def _gather_call(table, edges_flat):
    return pl.kernel(
        _gather_body,
        out_type=jax.ShapeDtypeStruct((B * M, D), jnp.float32),
        mesh=_sc_mesh(),
        scratch_types=[
            pltpu.VMEM((2 * CE,), jnp.int32),
            pltpu.VMEM((2 * CE,), jnp.int32),
            pltpu.VMEM((2, CE), jnp.int32),
            pltpu.VMEM((2, CE), jnp.int32),
            pltpu.VMEM((CE, D), jnp.float32),
            pltpu.VMEM((CE, D), jnp.float32),
            pltpu.VMEM((CE, D), jnp.float32),
            pltpu.VMEM((CE, D), jnp.float32),
            pltpu.VMEM((CE, D), jnp.float32),
            pltpu.SemaphoreType.DMA,
            pltpu.SemaphoreType.DMA,
        ],
    )(table, edges_flat)


def _scatter_call(ee, edges_flat, z64):
    return pl.kernel(
        _scatter_body,
        out_type=jax.ShapeDtypeStruct((2 * B * NP1, D), jnp.float32),
        mesh=_sc_mesh(),
        scratch_types=[
            pltpu.VMEM_SHARED((NP1, D), jnp.float32),
            pltpu.VMEM((CH, D), jnp.float32),
            pltpu.VMEM((CH,), jnp.int32),
            pltpu.VMEM((BCH, D), jnp.float32),
            pltpu.VMEM((BCH, D), jnp.float32),
        ],
    )(ee, edges_flat, z64)


def kernel(V, E, edges, fe_W1, fe_b1, fe_W2, fe_b2,
           fn_W1, fn_b1, fn_W2, fn_b2):
    f32 = jnp.float32
    w_sr = fe_W1[:256].reshape(2, D, D)
    w1e = fe_W1[256:]
    b1e = fe_b1.reshape(1, D)
    b2e = fe_b2.reshape(1, D)
    wv, wm0, wm1 = fn_W1[:128], fn_W1[128:192], fn_W1[192:256]
    b1n = fn_b1.reshape(1, D)
    b2n = fn_b2.reshape(1, D)

    tn = 2000
    table = pl.pallas_call(
        _pre_body,
        grid=(B, 2, N // tn),
        in_specs=[
            pl.BlockSpec((1, tn, D), lambda b, p, i: (b, i, 0)),
            pl.BlockSpec((1, D, D), lambda b, p, i: (p, 0, 0)),
        ],
        out_specs=pl.BlockSpec((1, 1, tn, D), lambda b, p, i: (b, p, i, 0)),
        out_shape=jax.ShapeDtypeStruct((B, 2, N, D), f32),
    )(V, w_sr).reshape(B * 2 * N, D)

    edges_flat = edges.reshape(B * 2 * M)

    G = _gather_call(table, edges_flat)

    tm = 1024
    EE = pl.pallas_call(
        _edge_body,
        grid=(B * M // tm,),
        in_specs=[
            pl.BlockSpec((tm, D), lambda i: (i, 0)),
            pl.BlockSpec((tm, D), lambda i: (i, 0)),
            pl.BlockSpec((D, D), lambda i: (0, 0)),
            pl.BlockSpec((1, D), lambda i: (0, 0)),
            pl.BlockSpec((D, D), lambda i: (0, 0)),
            pl.BlockSpec((1, D), lambda i: (0, 0)),
        ],
        out_specs=pl.BlockSpec((tm, D), lambda i: (i, 0)),
        out_shape=jax.ShapeDtypeStruct((B * M, D), f32),
    )(G, E.reshape(B * M, D), w1e, b1e, fe_W2, b2e)

    z64 = jnp.zeros((BCH, D), f32)
    edges_t = edges.transpose(0, 2, 1).reshape(B * 2 * M)
    accs = _scatter_call(EE, edges_t, z64).reshape(2, B, NP1, D)

    node_emb = _node_call(V, accs, wv, wm0, wm1, b1n, fn_W2, b2n)

    return (node_emb, EE.reshape(B, M, D))


def _node_call(V, accs, wv, wm0, wm1, b1n, w2, b2n):
    tno = 1000
    return pl.pallas_call(
        _node_body,
        grid=(B, N // tno),
        in_specs=[
            pl.BlockSpec((1, tno, D), lambda b, i: (b, i, 0)),
            pl.BlockSpec((1, 1, tno, D), lambda b, i: (0, b, i, 0)),
            pl.BlockSpec((1, 1, tno, D), lambda b, i: (1, b, i, 0)),
            pl.BlockSpec((D, D), lambda b, i: (0, 0)),
            pl.BlockSpec((64, D), lambda b, i: (0, 0)),
            pl.BlockSpec((64, D), lambda b, i: (0, 0)),
            pl.BlockSpec((1, D), lambda b, i: (0, 0)),
            pl.BlockSpec((D, D), lambda b, i: (0, 0)),
            pl.BlockSpec((1, D), lambda b, i: (0, 0)),
        ],
        out_specs=pl.BlockSpec((1, tno, D), lambda b, i: (b, i, 0)),
        out_shape=jax.ShapeDtypeStruct((B, N, D), jnp.float32),
    )(V, accs, accs, wv, wm0, wm1, b1n, w2, b2n)


# speedup vs baseline: 9.9322x; 1.2166x over previous
import functools

import jax
import jax.numpy as jnp
from jax import lax
from jax.experimental import pallas as pl
from jax.experimental.pallas import tpu as pltpu
from jax.experimental.pallas import tpu_sc as plsc

B, N, M = 4, 10000, 160000
D = 128
NW = 32
EPW = (B * M) // NW
CE = 80
NCH = EPW // CE
CR = 160
RPS = (2 * M) // 16
NSC = RPS // CR
NP = 20480
PSUB = NP // 16


def _pre_body(v_ref, w_ref, o_ref):
    o_ref[0, 0] = jnp.dot(v_ref[0], w_ref[0],
                          preferred_element_type=jnp.float32)


def _edge_body(g_ref, e_ref, w1_ref, b1_ref, w2_ref, b2_ref, o_ref):
    x = g_ref[...] + jnp.dot(e_ref[...], w1_ref[...],
                             preferred_element_type=jnp.float32) + b1_ref[...]
    h = x * jax.nn.sigmoid(x)
    o_ref[...] = jnp.dot(h, w2_ref[...],
                         preferred_element_type=jnp.float32) + b2_ref[...]


def _node_body(v_ref, s0_ref, s1_ref, wv_ref, wm0_ref,
               wm1_ref, b1_ref, w2_ref, b2_ref, o_ref):
    s0 = s0_ref[0, 0]
    s1 = s1_ref[0, 0]
    m0 = s0[:, :64] / jnp.maximum(s0[:, 64:65], 1.0)
    m1 = s1[:, 64:] / jnp.maximum(s1[:, 0:1], 1.0)
    x = (jnp.dot(v_ref[0], wv_ref[...], preferred_element_type=jnp.float32)
         + jnp.dot(m0, wm0_ref[...], preferred_element_type=jnp.float32)
         + jnp.dot(m1, wm1_ref[...], preferred_element_type=jnp.float32)
         + b1_ref[...])
    h = x * jax.nn.sigmoid(x)
    o_ref[0] = jnp.dot(h, w2_ref[...],
                       preferred_element_type=jnp.float32) + b2_ref[...]


def _gather_body(table, edges, out, ebuf0, ebuf1, idx0, idx1,
                 ga0, gb0, ga1, gb1, obuf, sem0, sem1):
    c = lax.axis_index("c")
    s = lax.axis_index("s")
    wid = c * 16 + s
    e_base = wid * EPW
    b = e_base // M
    off = (lax.iota(jnp.int32, 16) & 1) * N + b * (2 * N)

    def stage(i, ebuf, idx2, g_a, g_b, sem):
        be = e_base + i * CE
        pltpu.sync_copy(edges.at[pl.ds(2 * be, 2 * CE)], ebuf)
        for t in range(2 * CE // 16):
            v = ebuf[pl.ds(t * 16, 16)] + off
            idx2[(t * 16) // CE, pl.ds((t * 16) % CE, 16)] = v
        pltpu.async_copy(table.at[idx2.at[0]], g_a, sem)
        pltpu.async_copy(table.at[idx2.at[1]], g_b, sem)

    def drain(i, g_a, g_b, sem):
        be = e_base + i * CE
        pltpu.make_async_copy(table.at[idx0.at[0]], g_a, sem).wait()
        pltpu.make_async_copy(table.at[idx0.at[0]], g_b, sem).wait()

        def pair_add(j, carry2):
            for k in range(8):
                sl = pl.ds(k * 16, 16)
                obuf[j, sl] = g_a[2 * j, sl] + g_a[2 * j + 1, sl]
                obuf[CE // 2 + j, sl] = g_b[2 * j, sl] + g_b[2 * j + 1, sl]
            return carry2

        lax.fori_loop(0, CE // 2, pair_add, 0, unroll=4)
        pltpu.sync_copy(obuf, out.at[pl.ds(be, CE)])

    stage(0, ebuf0, idx0, ga0, gb0, sem0)

    def body(k, carry):
        i0 = 2 * k
        stage(i0 + 1, ebuf1, idx1, ga1, gb1, sem1)
        drain(i0, ga0, gb0, sem0)

        @pl.when(k + 1 < NCH // 2)
        def _():
            stage(i0 + 2, ebuf0, idx0, ga0, gb0, sem0)

        drain(i0 + 1, ga1, gb1, sem1)
        return carry

    lax.fori_loop(0, NCH // 2, body, 0)


BCH = 64
NP1 = 10240
PS1 = NP1 // 16
CH = 80
NCH2 = (M // 16) // CH


def _scatter_body(ee, edges_t, z64, out, acc, vb0, vb1, ib0, ib1, zb, ob_buf,
                  sem0, sem1):
    c = lax.axis_index("c")
    s = lax.axis_index("s")
    coff = (1 - c) * 64
    onehot = jnp.where(lax.iota(jnp.int32, 16) == 0,
                       jnp.float32(1.0), jnp.float32(0.0))
    pltpu.sync_copy(z64, zb)
    for b in range(B):

        def zero_k(k, carry):
            pltpu.sync_copy(zb, acc.at[pl.ds(s * PS1 + k * BCH, BCH)])
            return carry

        lax.fori_loop(0, PS1 // BCH, zero_k, 0)
        plsc.subcore_barrier()
        eb0 = b * M + s * (M // 16)
        tb0 = (2 * b + c) * M + s * (M // 16)

        def load(i, vb, ib, sem):
            pltpu.async_copy(ee.at[pl.ds(eb0 + i * CH, CH)], vb, sem)
            pltpu.async_copy(edges_t.at[pl.ds(tb0 + i * CH, CH)], ib, sem)

        def proc(vb, ib, sem):
            pltpu.make_async_copy(ee.at[pl.ds(eb0, CH)], vb, sem).wait()
            pltpu.make_async_copy(edges_t.at[pl.ds(tb0, CH)], ib, sem).wait()

            def mark(j, carry2):
                for u in range(8):
                    vb[j * 8 + u, pl.ds(coff, 16)] = onehot
                return carry2

            lax.fori_loop(0, CH // 8, mark, 0, unroll=2)
            pltpu.sync_copy(vb, acc.at[ib], add=True)

        load(0, vb0, ib0, sem0)

        def body(k, carry):
            load(2 * k + 1, vb1, ib1, sem1)
            proc(vb0, ib0, sem0)
            load(2 * k + 2, vb0, ib0, sem0)
            proc(vb1, ib1, sem1)
            return carry

        lax.fori_loop(0, NCH2 // 2, body, 0)
        proc(vb0, ib0, sem0)
        plsc.subcore_barrier()
        base_o = (c * B + b) * NP1 + s * PS1

        def out_k(k, carry):
            pltpu.sync_copy(acc.at[pl.ds(s * PS1 + k * BCH, BCH)], ob_buf)
            pltpu.sync_copy(ob_buf, out.at[pl.ds(base_o + k * BCH, BCH)])
            return carry

        lax.fori_loop(0, PS1 // BCH, out_k, 0)
        plsc.subcore_barrier()


def _sc_mesh():
    return plsc.VectorSubcoreMesh(core_axis_name="c", subcore_axis_name="s")


def _gather_call(table, edges_flat):
    return pl.kernel(
        _gather_body,
        out_type=jax.ShapeDtypeStruct((B * M, D), jnp.float32),
        mesh=_sc_mesh(),
        scratch_types=[
            pltpu.VMEM((2 * CE,), jnp.int32),
            pltpu.VMEM((2 * CE,), jnp.int32),
            pltpu.VMEM((2, CE), jnp.int32),
            pltpu.VMEM((2, CE), jnp.int32),
            pltpu.VMEM((CE, D), jnp.float32),
            pltpu.VMEM((CE, D), jnp.float32),
            pltpu.VMEM((CE, D), jnp.float32),
            pltpu.VMEM((CE, D), jnp.float32),
            pltpu.VMEM((CE, D), jnp.float32),
            pltpu.SemaphoreType.DMA,
            pltpu.SemaphoreType.DMA,
        ],
    )(table, edges_flat)


def _scatter_call(ee, edges_flat, z64):
    return pl.kernel(
        _scatter_body,
        out_type=jax.ShapeDtypeStruct((2 * B * NP1, D), jnp.float32),
        mesh=_sc_mesh(),
        scratch_types=[
            pltpu.VMEM_SHARED((NP1, D), jnp.float32),
            pltpu.VMEM((CH, D), jnp.float32),
            pltpu.VMEM((CH, D), jnp.float32),
            pltpu.VMEM((CH,), jnp.int32),
            pltpu.VMEM((CH,), jnp.int32),
            pltpu.VMEM((BCH, D), jnp.float32),
            pltpu.VMEM((BCH, D), jnp.float32),
            pltpu.SemaphoreType.DMA,
            pltpu.SemaphoreType.DMA,
        ],
    )(ee, edges_flat, z64)


def kernel(V, E, edges, fe_W1, fe_b1, fe_W2, fe_b2,
           fn_W1, fn_b1, fn_W2, fn_b2):
    f32 = jnp.float32
    w_sr = fe_W1[:256].reshape(2, D, D)
    w1e = fe_W1[256:]
    b1e = fe_b1.reshape(1, D)
    b2e = fe_b2.reshape(1, D)
    wv, wm0, wm1 = fn_W1[:128], fn_W1[128:192], fn_W1[192:256]
    b1n = fn_b1.reshape(1, D)
    b2n = fn_b2.reshape(1, D)

    tn = 2000
    table = pl.pallas_call(
        _pre_body,
        grid=(B, 2, N // tn),
        in_specs=[
            pl.BlockSpec((1, tn, D), lambda b, p, i: (b, i, 0)),
            pl.BlockSpec((1, D, D), lambda b, p, i: (p, 0, 0)),
        ],
        out_specs=pl.BlockSpec((1, 1, tn, D), lambda b, p, i: (b, p, i, 0)),
        out_shape=jax.ShapeDtypeStruct((B, 2, N, D), f32),
    )(V, w_sr).reshape(B * 2 * N, D)

    edges_flat = edges.reshape(B * 2 * M)

    G = _gather_call(table, edges_flat)

    tm = 1024
    EE = pl.pallas_call(
        _edge_body,
        grid=(B * M // tm,),
        in_specs=[
            pl.BlockSpec((tm, D), lambda i: (i, 0)),
            pl.BlockSpec((tm, D), lambda i: (i, 0)),
            pl.BlockSpec((D, D), lambda i: (0, 0)),
            pl.BlockSpec((1, D), lambda i: (0, 0)),
            pl.BlockSpec((D, D), lambda i: (0, 0)),
            pl.BlockSpec((1, D), lambda i: (0, 0)),
        ],
        out_specs=pl.BlockSpec((tm, D), lambda i: (i, 0)),
        out_shape=jax.ShapeDtypeStruct((B * M, D), f32),
    )(G, E.reshape(B * M, D), w1e, b1e, fe_W2, b2e)

    z64 = jnp.zeros((BCH, D), f32)
    edges_t = edges.transpose(0, 2, 1).reshape(B * 2 * M)
    accs = _scatter_call(EE, edges_t, z64).reshape(2, B, NP1, D)

    node_emb = _node_call(V, accs, wv, wm0, wm1, b1n, fn_W2, b2n)

    return (node_emb, EE.reshape(B, M, D))


def _node_call(V, accs, wv, wm0, wm1, b1n, w2, b2n):
    tno = 1000
    return pl.pallas_call(
        _node_body,
        grid=(B, N // tno),
        in_specs=[
            pl.BlockSpec((1, tno, D), lambda b, i: (b, i, 0)),
            pl.BlockSpec((1, 1, tno, D), lambda b, i: (0, b, i, 0)),
            pl.BlockSpec((1, 1, tno, D), lambda b, i: (1, b, i, 0)),
            pl.BlockSpec((D, D), lambda b, i: (0, 0)),
            pl.BlockSpec((64, D), lambda b, i: (0, 0)),
            pl.BlockSpec((64, D), lambda b, i: (0, 0)),
            pl.BlockSpec((1, D), lambda b, i: (0, 0)),
            pl.BlockSpec((D, D), lambda b, i: (0, 0)),
            pl.BlockSpec((1, D), lambda b, i: (0, 0)),
        ],
        out_specs=pl.BlockSpec((1, tno, D), lambda b, i: (b, i, 0)),
        out_shape=jax.ShapeDtypeStruct((B, N, D), jnp.float32),
    )(V, accs, accs, wv, wm0, wm1, b1n, w2, b2n)


# speedup vs baseline: 12.1271x; 1.2210x over previous
import functools

import jax
import jax.numpy as jnp
from jax import lax
from jax.experimental import pallas as pl
from jax.experimental.pallas import tpu as pltpu
from jax.experimental.pallas import tpu_sc as plsc

B, N, M = 4, 10000, 160000
D = 128
NW = 32
EPW = (B * M) // NW
CE = 80
NCH = EPW // CE
CR = 160
RPS = (2 * M) // 16
NSC = RPS // CR
NP = 20480
PSUB = NP // 16


def _pre_body(v_ref, w_ref, o_ref):
    o_ref[0, 0] = jnp.dot(v_ref[0], w_ref[0],
                          preferred_element_type=jnp.float32)


def _edge_body(g_ref, e_ref, w1_ref, b1_ref, w2_ref, b2_ref, o_ref):
    x = g_ref[...] + jnp.dot(e_ref[...], w1_ref[...],
                             preferred_element_type=jnp.float32) + b1_ref[...]
    h = x * jax.nn.sigmoid(x)
    o_ref[...] = jnp.dot(h, w2_ref[...],
                         preferred_element_type=jnp.float32) + b2_ref[...]


def _node_body(v_ref, s0_ref, s1_ref, wv_ref, wm0_ref,
               wm1_ref, b1_ref, w2_ref, b2_ref, o_ref):
    s0 = s0_ref[0, 0]
    s1 = s1_ref[0, 0]
    m0 = s0[:, :64] / jnp.maximum(s0[:, 64:65], 1.0)
    m1 = s1[:, 64:] / jnp.maximum(s1[:, 0:1], 1.0)
    x = (jnp.dot(v_ref[0], wv_ref[...], preferred_element_type=jnp.float32)
         + jnp.dot(m0, wm0_ref[...], preferred_element_type=jnp.float32)
         + jnp.dot(m1, wm1_ref[...], preferred_element_type=jnp.float32)
         + b1_ref[...])
    h = x * jax.nn.sigmoid(x)
    o_ref[0] = jnp.dot(h, w2_ref[...],
                       preferred_element_type=jnp.float32) + b2_ref[...]


def _gather_body(table, edges_t, out, sbig, rbig, idx0, idx1,
                 ga0, gb0, ga1, gb1, obuf, sem0, sem1):
    c = lax.axis_index("c")
    s = lax.axis_index("s")
    wid = c * 16 + s
    e_base = wid * EPW
    b = e_base // M
    boff = b * (2 * N)
    sb0 = b * 2 * M + (e_base - b * M)
    pltpu.sync_copy(edges_t.at[pl.ds(sb0, EPW)], sbig)
    pltpu.sync_copy(edges_t.at[pl.ds(sb0 + M, EPW)], rbig)

    def stage(i, idx2, g_a, g_b, sem):
        for t in range(CE // 16):
            idx2[0, pl.ds(t * 16, 16)] = (
                sbig[pl.ds(i * CE + t * 16, 16)] + boff)
            idx2[1, pl.ds(t * 16, 16)] = (
                rbig[pl.ds(i * CE + t * 16, 16)] + (N + boff))
        pltpu.async_copy(table.at[idx2.at[0]], g_a, sem)
        pltpu.async_copy(table.at[idx2.at[1]], g_b, sem)

    def drain(i, g_a, g_b, sem):
        be = e_base + i * CE
        pltpu.make_async_copy(table.at[idx0.at[0]], g_a, sem).wait()
        pltpu.make_async_copy(table.at[idx0.at[0]], g_b, sem).wait()

        def pair_add(j, carry2):
            for k in range(8):
                sl = pl.ds(k * 16, 16)
                obuf[j, sl] = g_a[j, sl] + g_b[j, sl]
            return carry2

        lax.fori_loop(0, CE, pair_add, 0, unroll=4)
        pltpu.sync_copy(obuf, out.at[pl.ds(be, CE)])

    stage(0, idx0, ga0, gb0, sem0)

    def body(k, carry):
        i0 = 2 * k
        stage(i0 + 1, idx1, ga1, gb1, sem1)
        drain(i0, ga0, gb0, sem0)

        @pl.when(k + 1 < NCH // 2)
        def _():
            stage(i0 + 2, idx0, ga0, gb0, sem0)

        drain(i0 + 1, ga1, gb1, sem1)
        return carry

    lax.fori_loop(0, NCH // 2, body, 0)


BCH = 64
NP1 = 10240
PS1 = NP1 // 16
CH = 80
NCH2 = (M // 16) // CH


def _scatter_body(ee, edges_t, z64, out, acc, vb0, vb1, ib0, ib1, zb, ob_buf,
                  sem0, sem1):
    c = lax.axis_index("c")
    s = lax.axis_index("s")
    coff = (1 - c) * 64
    onehot = jnp.where(lax.iota(jnp.int32, 16) == 0,
                       jnp.float32(1.0), jnp.float32(0.0))
    pltpu.sync_copy(z64, zb)
    for b in range(B):

        def zero_k(k, carry):
            pltpu.sync_copy(zb, acc.at[pl.ds(s * PS1 + k * BCH, BCH)])
            return carry

        lax.fori_loop(0, PS1 // BCH, zero_k, 0)
        plsc.subcore_barrier()
        eb0 = b * M + s * (M // 16)
        tb0 = (2 * b + c) * M + s * (M // 16)

        def load(i, vb, ib, sem):
            pltpu.async_copy(ee.at[pl.ds(eb0 + i * CH, CH)], vb, sem)
            pltpu.async_copy(edges_t.at[pl.ds(tb0 + i * CH, CH)], ib, sem)

        def proc(vb, ib, sem):
            pltpu.make_async_copy(ee.at[pl.ds(eb0, CH)], vb, sem).wait()
            pltpu.make_async_copy(edges_t.at[pl.ds(tb0, CH)], ib, sem).wait()

            def mark(j, carry2):
                for u in range(8):
                    vb[j * 8 + u, pl.ds(coff, 16)] = onehot
                return carry2

            lax.fori_loop(0, CH // 8, mark, 0, unroll=2)
            pltpu.sync_copy(vb, acc.at[ib], add=True)

        load(0, vb0, ib0, sem0)

        def body(k, carry):
            load(2 * k + 1, vb1, ib1, sem1)
            proc(vb0, ib0, sem0)
            load(2 * k + 2, vb0, ib0, sem0)
            proc(vb1, ib1, sem1)
            return carry

        lax.fori_loop(0, NCH2 // 2, body, 0)
        proc(vb0, ib0, sem0)
        plsc.subcore_barrier()
        base_o = (c * B + b) * NP1 + s * PS1

        def out_k(k, carry):
            pltpu.sync_copy(acc.at[pl.ds(s * PS1 + k * BCH, BCH)], ob_buf)
            pltpu.sync_copy(ob_buf, out.at[pl.ds(base_o + k * BCH, BCH)])
            return carry

        lax.fori_loop(0, PS1 // BCH, out_k, 0)
        plsc.subcore_barrier()


def _sc_mesh():
    return plsc.VectorSubcoreMesh(core_axis_name="c", subcore_axis_name="s")


def _gather_call(table, edges_flat):
    return pl.kernel(
        _gather_body,
        out_type=jax.ShapeDtypeStruct((B * M, D), jnp.float32),
        mesh=_sc_mesh(),
        scratch_types=[
            pltpu.VMEM((EPW,), jnp.int32),
            pltpu.VMEM((EPW,), jnp.int32),
            pltpu.VMEM((2, CE), jnp.int32),
            pltpu.VMEM((2, CE), jnp.int32),
            pltpu.VMEM((CE, D), jnp.float32),
            pltpu.VMEM((CE, D), jnp.float32),
            pltpu.VMEM((CE, D), jnp.float32),
            pltpu.VMEM((CE, D), jnp.float32),
            pltpu.VMEM((CE, D), jnp.float32),
            pltpu.SemaphoreType.DMA,
            pltpu.SemaphoreType.DMA,
        ],
    )(table, edges_flat)


def _scatter_call(ee, edges_flat, z64):
    return pl.kernel(
        _scatter_body,
        out_type=jax.ShapeDtypeStruct((2 * B * NP1, D), jnp.float32),
        mesh=_sc_mesh(),
        scratch_types=[
            pltpu.VMEM_SHARED((NP1, D), jnp.float32),
            pltpu.VMEM((CH, D), jnp.float32),
            pltpu.VMEM((CH, D), jnp.float32),
            pltpu.VMEM((CH,), jnp.int32),
            pltpu.VMEM((CH,), jnp.int32),
            pltpu.VMEM((BCH, D), jnp.float32),
            pltpu.VMEM((BCH, D), jnp.float32),
            pltpu.SemaphoreType.DMA,
            pltpu.SemaphoreType.DMA,
        ],
    )(ee, edges_flat, z64)


def kernel(V, E, edges, fe_W1, fe_b1, fe_W2, fe_b2,
           fn_W1, fn_b1, fn_W2, fn_b2):
    f32 = jnp.float32
    w_sr = fe_W1[:256].reshape(2, D, D)
    w1e = fe_W1[256:]
    b1e = fe_b1.reshape(1, D)
    b2e = fe_b2.reshape(1, D)
    wv, wm0, wm1 = fn_W1[:128], fn_W1[128:192], fn_W1[192:256]
    b1n = fn_b1.reshape(1, D)
    b2n = fn_b2.reshape(1, D)

    tn = 2000
    table = pl.pallas_call(
        _pre_body,
        grid=(B, 2, N // tn),
        in_specs=[
            pl.BlockSpec((1, tn, D), lambda b, p, i: (b, i, 0)),
            pl.BlockSpec((1, D, D), lambda b, p, i: (p, 0, 0)),
        ],
        out_specs=pl.BlockSpec((1, 1, tn, D), lambda b, p, i: (b, p, i, 0)),
        out_shape=jax.ShapeDtypeStruct((B, 2, N, D), jnp.float32),
    )(V, w_sr).reshape(B * 2 * N, D)

    edges_t = edges.transpose(0, 2, 1).reshape(B * 2 * M)

    G = _gather_call(table, edges_t)

    tm = 1024
    EE = pl.pallas_call(
        _edge_body,
        grid=(B * M // tm,),
        in_specs=[
            pl.BlockSpec((tm, D), lambda i: (i, 0)),
            pl.BlockSpec((tm, D), lambda i: (i, 0)),
            pl.BlockSpec((D, D), lambda i: (0, 0)),
            pl.BlockSpec((1, D), lambda i: (0, 0)),
            pl.BlockSpec((D, D), lambda i: (0, 0)),
            pl.BlockSpec((1, D), lambda i: (0, 0)),
        ],
        out_specs=pl.BlockSpec((tm, D), lambda i: (i, 0)),
        out_shape=jax.ShapeDtypeStruct((B * M, D), f32),
    )(G, E.reshape(B * M, D), w1e, b1e, fe_W2, b2e)

    z64 = jnp.zeros((BCH, D), f32)
    accs = _scatter_call(EE, edges_t, z64).reshape(2, B, NP1, D)

    node_emb = _node_call(V, accs, wv, wm0, wm1, b1n, fn_W2, b2n)

    return (node_emb, EE.reshape(B, M, D))


def _node_call(V, accs, wv, wm0, wm1, b1n, w2, b2n):
    tno = 1000
    return pl.pallas_call(
        _node_body,
        grid=(B, N // tno),
        in_specs=[
            pl.BlockSpec((1, tno, D), lambda b, i: (b, i, 0)),
            pl.BlockSpec((1, 1, tno, D), lambda b, i: (0, b, i, 0)),
            pl.BlockSpec((1, 1, tno, D), lambda b, i: (1, b, i, 0)),
            pl.BlockSpec((D, D), lambda b, i: (0, 0)),
            pl.BlockSpec((64, D), lambda b, i: (0, 0)),
            pl.BlockSpec((64, D), lambda b, i: (0, 0)),
            pl.BlockSpec((1, D), lambda b, i: (0, 0)),
            pl.BlockSpec((D, D), lambda b, i: (0, 0)),
            pl.BlockSpec((1, D), lambda b, i: (0, 0)),
        ],
        out_specs=pl.BlockSpec((1, tno, D), lambda b, i: (b, i, 0)),
        out_shape=jax.ShapeDtypeStruct((B, N, D), jnp.float32),
    )(V, accs, accs, wv, wm0, wm1, b1n, w2, b2n)


# speedup vs baseline: 12.6469x; 1.0429x over previous
import functools

import jax
import jax.numpy as jnp
from jax import lax
from jax.experimental import pallas as pl
from jax.experimental.pallas import tpu as pltpu
from jax.experimental.pallas import tpu_sc as plsc

B, N, M = 4, 10000, 160000
D = 128
NW = 32
EPW = (B * M) // NW
CE = 80
NCH = EPW // CE
CR = 160
RPS = (2 * M) // 16
NSC = RPS // CR
NP = 20480
PSUB = NP // 16


def _pre_body(v_ref, w_ref, o_ref):
    o_ref[0, 0] = jnp.dot(v_ref[0], w_ref[0],
                          preferred_element_type=jnp.float32)


def _edge_body(g_ref, e_ref, w1_ref, b1_ref, w2_ref, b2_ref, o_ref):
    x = g_ref[...] + jnp.dot(e_ref[...], w1_ref[...],
                             preferred_element_type=jnp.float32) + b1_ref[...]
    h = x * jax.nn.sigmoid(x)
    o_ref[...] = jnp.dot(h, w2_ref[...],
                         preferred_element_type=jnp.float32) + b2_ref[...]


def _node_body(v_ref, s0_ref, s1_ref, wv_ref, wm0_ref,
               wm1_ref, b1_ref, w2_ref, b2_ref, o_ref):
    s0 = s0_ref[0, 0]
    s1 = s1_ref[0, 0]
    m0 = s0[:, :64] / jnp.maximum(s0[:, 64:65], 1.0)
    m1 = s1[:, 64:] / jnp.maximum(s1[:, 0:1], 1.0)
    x = (jnp.dot(v_ref[0], wv_ref[...], preferred_element_type=jnp.float32)
         + jnp.dot(m0, wm0_ref[...], preferred_element_type=jnp.float32)
         + jnp.dot(m1, wm1_ref[...], preferred_element_type=jnp.float32)
         + b1_ref[...])
    h = x * jax.nn.sigmoid(x)
    o_ref[0] = jnp.dot(h, w2_ref[...],
                       preferred_element_type=jnp.float32) + b2_ref[...]


def _gather_body(table, edges_t, out, sbig, rbig, idx0, idx1, idx2b, idx3b,
                 ga0, gb0, ga1, gb1, ga2, gb2, ga3, gb3,
                 sem0, sem1, sem2, sem3):
    c = lax.axis_index("c")
    s = lax.axis_index("s")
    wid = c * 16 + s
    e_base = wid * EPW
    b = e_base // M
    boff = b * (2 * N)
    sb0 = b * 2 * M + (e_base - b * M)
    pltpu.sync_copy(edges_t.at[pl.ds(sb0, EPW)], sbig)
    pltpu.sync_copy(edges_t.at[pl.ds(sb0 + M, EPW)], rbig)

    sets = ((idx0, ga0, gb0, sem0), (idx1, ga1, gb1, sem1),
            (idx2b, ga2, gb2, sem2), (idx3b, ga3, gb3, sem3))

    def stage(i, st):
        idx2, g_a, g_b, sem = st
        for t in range(CE // 16):
            idx2[0, pl.ds(t * 16, 16)] = (
                sbig[pl.ds(i * CE + t * 16, 16)] + boff)
            idx2[1, pl.ds(t * 16, 16)] = (
                rbig[pl.ds(i * CE + t * 16, 16)] + (N + boff))
        pltpu.async_copy(table.at[idx2.at[0]], g_a, sem)
        pltpu.async_copy(table.at[idx2.at[1]], g_b, sem)

    def drain(i, st):
        idx2, g_a, g_b, sem = st
        be = e_base + i * CE
        pltpu.make_async_copy(table.at[idx2.at[0]], g_a, sem).wait()
        pltpu.make_async_copy(table.at[idx2.at[0]], g_b, sem).wait()

        def pair_add(j, carry2):
            for k in range(8):
                sl = pl.ds(k * 16, 16)
                g_a[j, sl] = g_a[j, sl] + g_b[j, sl]
            return carry2

        lax.fori_loop(0, CE, pair_add, 0, unroll=4)
        pltpu.sync_copy(g_a, out.at[pl.ds(be, CE)])

    stage(0, sets[0])
    stage(1, sets[1])

    def body(k, carry):
        i0 = 4 * k
        stage(i0 + 2, sets[2])
        drain(i0, sets[0])
        stage(i0 + 3, sets[3])
        drain(i0 + 1, sets[1])
        stage(i0 + 4, sets[0])
        drain(i0 + 2, sets[2])
        stage(i0 + 5, sets[1])
        drain(i0 + 3, sets[3])
        return carry

    lax.fori_loop(0, (NCH - 2) // 4, body, 0)
    drain(NCH - 2, sets[0])
    drain(NCH - 1, sets[1])


BCH = 64
NP1 = 10240
PS1 = NP1 // 16
CH = 80
NCH2 = (M // 16) // CH


def _scatter_body(ee, edges_t, z64, out, acc, vb0, vb1, ib0, ib1, zb, ob_buf,
                  sem0, sem1):
    c = lax.axis_index("c")
    s = lax.axis_index("s")
    coff = (1 - c) * 64
    onehot = jnp.where(lax.iota(jnp.int32, 16) == 0,
                       jnp.float32(1.0), jnp.float32(0.0))
    pltpu.sync_copy(z64, zb)
    for b in range(B):

        def zero_k(k, carry):
            pltpu.sync_copy(zb, acc.at[pl.ds(s * PS1 + k * BCH, BCH)])
            return carry

        lax.fori_loop(0, PS1 // BCH, zero_k, 0)
        plsc.subcore_barrier()
        eb0 = b * M + s * (M // 16)
        tb0 = (2 * b + c) * M + s * (M // 16)

        def load(i, vb, ib, sem):
            pltpu.async_copy(ee.at[pl.ds(eb0 + i * CH, CH)], vb, sem)
            pltpu.async_copy(edges_t.at[pl.ds(tb0 + i * CH, CH)], ib, sem)

        def proc(vb, ib, sem):
            pltpu.make_async_copy(ee.at[pl.ds(eb0, CH)], vb, sem).wait()
            pltpu.make_async_copy(edges_t.at[pl.ds(tb0, CH)], ib, sem).wait()

            def mark(j, carry2):
                for u in range(8):
                    vb[j * 8 + u, pl.ds(coff, 16)] = onehot
                return carry2

            lax.fori_loop(0, CH // 8, mark, 0, unroll=2)
            pltpu.sync_copy(vb, acc.at[ib], add=True)

        load(0, vb0, ib0, sem0)

        def body(k, carry):
            load(2 * k + 1, vb1, ib1, sem1)
            proc(vb0, ib0, sem0)
            load(2 * k + 2, vb0, ib0, sem0)
            proc(vb1, ib1, sem1)
            return carry

        lax.fori_loop(0, NCH2 // 2, body, 0)
        proc(vb0, ib0, sem0)
        plsc.subcore_barrier()
        base_o = (c * B + b) * NP1 + s * PS1

        def out_k(k, carry):
            pltpu.sync_copy(acc.at[pl.ds(s * PS1 + k * BCH, BCH)], ob_buf)
            pltpu.sync_copy(ob_buf, out.at[pl.ds(base_o + k * BCH, BCH)])
            return carry

        lax.fori_loop(0, PS1 // BCH, out_k, 0)
        plsc.subcore_barrier()


def _sc_mesh():
    return plsc.VectorSubcoreMesh(core_axis_name="c", subcore_axis_name="s")


def _gather_call(table, edges_flat):
    return pl.kernel(
        _gather_body,
        out_type=jax.ShapeDtypeStruct((B * M, D), jnp.float32),
        mesh=_sc_mesh(),
        scratch_types=[
            pltpu.VMEM((EPW,), jnp.int32),
            pltpu.VMEM((EPW,), jnp.int32),
            pltpu.VMEM((2, CE), jnp.int32),
            pltpu.VMEM((2, CE), jnp.int32),
            pltpu.VMEM((2, CE), jnp.int32),
            pltpu.VMEM((2, CE), jnp.int32),
            pltpu.VMEM((CE, D), jnp.float32),
            pltpu.VMEM((CE, D), jnp.float32),
            pltpu.VMEM((CE, D), jnp.float32),
            pltpu.VMEM((CE, D), jnp.float32),
            pltpu.VMEM((CE, D), jnp.float32),
            pltpu.VMEM((CE, D), jnp.float32),
            pltpu.VMEM((CE, D), jnp.float32),
            pltpu.VMEM((CE, D), jnp.float32),
            pltpu.SemaphoreType.DMA,
            pltpu.SemaphoreType.DMA,
            pltpu.SemaphoreType.DMA,
            pltpu.SemaphoreType.DMA,
        ],
    )(table, edges_flat)


def _scatter_call(ee, edges_flat, z64):
    return pl.kernel(
        _scatter_body,
        out_type=jax.ShapeDtypeStruct((2 * B * NP1, D), jnp.float32),
        mesh=_sc_mesh(),
        scratch_types=[
            pltpu.VMEM_SHARED((NP1, D), jnp.float32),
            pltpu.VMEM((CH, D), jnp.float32),
            pltpu.VMEM((CH, D), jnp.float32),
            pltpu.VMEM((CH,), jnp.int32),
            pltpu.VMEM((CH,), jnp.int32),
            pltpu.VMEM((BCH, D), jnp.float32),
            pltpu.VMEM((BCH, D), jnp.float32),
            pltpu.SemaphoreType.DMA,
            pltpu.SemaphoreType.DMA,
        ],
    )(ee, edges_flat, z64)


def kernel(V, E, edges, fe_W1, fe_b1, fe_W2, fe_b2,
           fn_W1, fn_b1, fn_W2, fn_b2):
    f32 = jnp.float32
    w_sr = fe_W1[:256].reshape(2, D, D)
    w1e = fe_W1[256:]
    b1e = fe_b1.reshape(1, D)
    b2e = fe_b2.reshape(1, D)
    wv, wm0, wm1 = fn_W1[:128], fn_W1[128:192], fn_W1[192:256]
    b1n = fn_b1.reshape(1, D)
    b2n = fn_b2.reshape(1, D)

    tn = 2000
    table = pl.pallas_call(
        _pre_body,
        grid=(B, 2, N // tn),
        in_specs=[
            pl.BlockSpec((1, tn, D), lambda b, p, i: (b, i, 0)),
            pl.BlockSpec((1, D, D), lambda b, p, i: (p, 0, 0)),
        ],
        out_specs=pl.BlockSpec((1, 1, tn, D), lambda b, p, i: (b, p, i, 0)),
        out_shape=jax.ShapeDtypeStruct((B, 2, N, D), jnp.float32),
    )(V, w_sr).reshape(B * 2 * N, D)

    edges_t = edges.transpose(0, 2, 1).reshape(B * 2 * M)

    G = _gather_call(table, edges_t)

    tm = 1024
    EE = pl.pallas_call(
        _edge_body,
        grid=(B * M // tm,),
        in_specs=[
            pl.BlockSpec((tm, D), lambda i: (i, 0)),
            pl.BlockSpec((tm, D), lambda i: (i, 0)),
            pl.BlockSpec((D, D), lambda i: (0, 0)),
            pl.BlockSpec((1, D), lambda i: (0, 0)),
            pl.BlockSpec((D, D), lambda i: (0, 0)),
            pl.BlockSpec((1, D), lambda i: (0, 0)),
        ],
        out_specs=pl.BlockSpec((tm, D), lambda i: (i, 0)),
        out_shape=jax.ShapeDtypeStruct((B * M, D), f32),
    )(G, E.reshape(B * M, D), w1e, b1e, fe_W2, b2e)

    z64 = jnp.zeros((BCH, D), f32)
    accs = _scatter_call(EE, edges_t, z64).reshape(2, B, NP1, D)

    node_emb = _node_call(V, accs, wv, wm0, wm1, b1n, fn_W2, b2n)

    return (node_emb, EE.reshape(B, M, D))


def _node_call(V, accs, wv, wm0, wm1, b1n, w2, b2n):
    tno = 1000
    return pl.pallas_call(
        _node_body,
        grid=(B, N // tno),
        in_specs=[
            pl.BlockSpec((1, tno, D), lambda b, i: (b, i, 0)),
            pl.BlockSpec((1, 1, tno, D), lambda b, i: (0, b, i, 0)),
            pl.BlockSpec((1, 1, tno, D), lambda b, i: (1, b, i, 0)),
            pl.BlockSpec((D, D), lambda b, i: (0, 0)),
            pl.BlockSpec((64, D), lambda b, i: (0, 0)),
            pl.BlockSpec((64, D), lambda b, i: (0, 0)),
            pl.BlockSpec((1, D), lambda b, i: (0, 0)),
            pl.BlockSpec((D, D), lambda b, i: (0, 0)),
            pl.BlockSpec((1, D), lambda b, i: (0, 0)),
        ],
        out_specs=pl.BlockSpec((1, tno, D), lambda b, i: (b, i, 0)),
        out_shape=jax.ShapeDtypeStruct((B, N, D), jnp.float32),
    )(V, accs, accs, wv, wm0, wm1, b1n, w2, b2n)
